# baseline (speedup 1.0000x reference)
import functools

import jax
import jax.numpy as jnp
from jax import lax
from jax.experimental import pallas as pl
from jax.experimental.pallas import tpu as pltpu
from jax.experimental.pallas import tpu_sc as plsc

NUM_CORES = 2
NUM_SUBCORES = 16
LANES = 16
NBUF = 4


@functools.partial(jax.jit, static_argnames=("seq_len", "seqs_per_chunk"))
def _sc_embed(xf, table, pos_emb, *, seq_len, seqs_per_chunk):
    n = xf.shape[0]
    d = table.shape[1]
    nw = NUM_CORES * NUM_SUBCORES
    per_w = n // nw
    chunk = seqs_per_chunk * seq_len
    n_chunks = per_w // chunk
    assert per_w % chunk == 0 and n % nw == 0 and d % LANES == 0
    assert n_chunks % NBUF == 0 and n_chunks >= 2 * NBUF
    vregs_per_row = d // LANES

    mesh = plsc.VectorSubcoreMesh(
        core_axis_name="c", subcore_axis_name="s",
        num_cores=NUM_CORES, num_subcores=NUM_SUBCORES,
    )

    @functools.partial(
        pl.kernel,
        mesh=mesh,
        out_type=jax.ShapeDtypeStruct((n, d), jnp.float32),
        scratch_types=[
            [pltpu.VMEM((chunk,), jnp.int32) for _ in range(NBUF)],
            [pltpu.VMEM((chunk, d), jnp.float32) for _ in range(NBUF)],
            [pltpu.VMEM((chunk, d), jnp.float32) for _ in range(NBUF)],
            pltpu.VMEM((chunk, d), jnp.float32),
            [pltpu.SemaphoreType.DMA for _ in range(NBUF)],
            [pltpu.SemaphoreType.DMA for _ in range(NBUF)],
        ],
        compiler_params=pltpu.CompilerParams(use_tc_tiling_on_sc=False),
    )
    def k(x_hbm, table_hbm, pos_hbm, out_hbm, idx_v, rows_v, obuf_v, pos_c,
          gsem, osem):
        wid = lax.axis_index("s") * NUM_CORES + lax.axis_index("c")
        base = wid * per_w

        for s in range(seqs_per_chunk):
            pltpu.sync_copy(pos_hbm, pos_c.at[pl.ds(s * seq_len, seq_len)])

        def stage_and_gather(c, b):
            off = base + c * chunk
            pltpu.sync_copy(x_hbm.at[pl.ds(off, chunk)], idx_v[b])
            pltpu.async_copy(table_hbm.at[idx_v[b]], rows_v[b], gsem[b])

        def add_pos(b):
            def body(r, carry):
                for v in range(vregs_per_row):
                    sl = pl.ds(v * LANES, LANES)
                    obuf_v[b][r, sl] = rows_v[b][r, sl] + pos_c[r, sl]
                return carry
            lax.fori_loop(0, chunk, body, 0, unroll=4)

        def wait_gather(b):
            pltpu.make_async_copy(table_hbm.at[idx_v[b]], rows_v[b],
                                  gsem[b]).wait()

        def start_out(c, b):
            off = base + c * chunk
            pltpu.async_copy(obuf_v[b], out_hbm.at[pl.ds(off, chunk)], osem[b])

        def wait_out(c, b):
            off = base + c * chunk
            pltpu.make_async_copy(obuf_v[b], out_hbm.at[pl.ds(off, chunk)],
                                  osem[b]).wait()

        for b in range(NBUF):
            stage_and_gather(b, b)

        for c in range(n_chunks):
            b = c % NBUF
            wait_gather(b)
            if c >= NBUF:
                wait_out(c - NBUF, b)
            add_pos(b)
            if c + NBUF < n_chunks:
                stage_and_gather(c + NBUF, b)
            start_out(c, b)

        for c in range(n_chunks - NBUF, n_chunks):
            wait_out(c, c % NBUF)

    return k(xf, table, pos_emb)


def kernel(x, table, pos_emb):
    b, l = x.shape
    xf = x.reshape(-1).astype(jnp.int32)
    out = _sc_embed(xf, table, pos_emb[:l], seq_len=l, seqs_per_chunk=2)
    return out.reshape(b, l, table.shape[1])
